# baseline (speedup 1.0000x reference)
import functools

import jax
import jax.numpy as jnp
from jax import lax
from jax.experimental import pallas as pl
from jax.experimental.pallas import tpu as pltpu
from jax.experimental.pallas import tpu_sc as plsc

B, H, D = 32, 32, 128
KV = 8192
K = 2048
NLANE = 16
CHUNK = KV // NLANE
RADIX = 256
NBINS = RADIX * NLANE

T_BLK = 2048


def _scores_body(q_ref, w_ref, kv_ref, scr_ref, out_ref):
    b = pl.program_id(0)
    t = pl.program_id(1)
    q = q_ref[0]
    kv = kv_ref[0]
    scale = jnp.float32(D) ** -0.5
    logits = lax.dot_general(
        kv, q, (((1,), (1,)), ((), ())),
        preferred_element_type=jnp.float32)
    act = jnp.maximum(logits * scale, 0.0)
    w = w_ref[b][None, :]
    s = lax.dot_general(
        w, act, (((1,), (1,)), ((), ())),
        preferred_element_type=jnp.float32)
    sl = pl.ds(t * T_BLK, T_BLK)
    out_ref[0, 0, sl] = s[0] + scr_ref[0, 0, sl]


def _scores(query, weights, index_kv_cache, index_scratch):
    grid = (B, KV // T_BLK)
    out = pl.pallas_call(
        _scores_body,
        grid=grid,
        in_specs=[
            pl.BlockSpec((1, H, D), lambda b, t: (b, 0, 0)),
            pl.BlockSpec((B, H), lambda b, t: (0, 0)),
            pl.BlockSpec((1, T_BLK, D), lambda b, t: (b, t, 0)),
            pl.BlockSpec((1, 1, KV), lambda b, t: (b, 0, 0)),
        ],
        out_specs=pl.BlockSpec((1, 1, KV), lambda b, t: (b, 0, 0)),
        out_shape=jax.ShapeDtypeStruct((B, 1, KV), jnp.float32),
    )(query, weights, index_kv_cache, index_scratch.reshape(B, 1, KV))
    return out.reshape(B, KV)


def _topk_body(scores_hbm, lens_hbm, out_s_hbm, out_i_hbm,
               sc_v, key_a, key_b, idx_a, idx_b, hist, lens_v,
               outs_v, outi_v):
    wid = lax.axis_index("s") * 2 + lax.axis_index("c")
    pltpu.sync_copy(scores_hbm.at[wid], sc_v)
    pltpu.sync_copy(lens_hbm.at[wid], lens_v)

    iota = lax.iota(jnp.int32, NLANE)
    len_vec = jnp.maximum(lens_v[...], jnp.int32(K))

    @pl.loop(0, CHUNK)
    def _build(it):
        sl = pl.ds(it * NLANE, NLANE)
        bits = lax.bitcast_convert_type(sc_v[sl], jnp.int32)
        key_m = jnp.where(bits >= 0, bits ^ jnp.int32(-2**31), ~bits)
        invkey = ~key_m
        pos = it * NLANE + iota
        invkey = jnp.where(pos < len_vec, invkey, jnp.int32(-1))
        key_a[sl] = invkey
        idx_a[sl] = pos

    ones = jnp.ones((NLANE,), jnp.int32)
    lane_base = iota * CHUNK

    for p in range(4):
        shift = jnp.int32(8 * p)
        src_k, src_i = (key_a, idx_a) if p % 2 == 0 else (key_b, idx_b)
        dst_k, dst_i = (key_b, idx_b) if p % 2 == 0 else (key_a, idx_a)

        @pl.loop(0, RADIX)
        def _zero(i):
            hist[pl.ds(i * NLANE, NLANE)] = jnp.zeros((NLANE,), jnp.int32)

        @pl.loop(0, CHUNK)
        def _hist(it):
            k = plsc.load_gather(src_k, [lane_base + it])
            d = lax.shift_right_logical(k, shift) & jnp.int32(0xFF)
            plsc.addupdate_scatter(hist, [d * NLANE + iota], ones)

        def _prefix(i, carry):
            sl = pl.ds(i * NLANE, NLANE)
            c = hist[sl]
            incl = plsc.cumsum(c)
            hist[sl] = incl - c + carry
            return carry + jnp.sum(c)

        lax.fori_loop(0, RADIX, _prefix, jnp.int32(0))

        @pl.loop(0, CHUNK)
        def _permute(it):
            gidx = lane_base + it
            k = plsc.load_gather(src_k, [gidx])
            v = plsc.load_gather(src_i, [gidx])
            d = lax.shift_right_logical(k, shift) & jnp.int32(0xFF)
            cidx = d * NLANE + iota
            pos = plsc.load_gather(hist, [cidx])
            plsc.store_scatter(dst_k, [pos], k)
            plsc.store_scatter(dst_i, [pos], v)
            plsc.addupdate_scatter(hist, [cidx], ones)

    @pl.loop(0, K // NLANE)
    def _emit(it):
        sl = pl.ds(it * NLANE, NLANE)
        i_vec = idx_a[sl]
        outs_v[sl] = plsc.load_gather(sc_v, [i_vec])
        outi_v[sl] = i_vec

    pltpu.sync_copy(outs_v, out_s_hbm.at[wid])
    pltpu.sync_copy(outi_v, out_i_hbm.at[wid])


def _topk(scores, kv_lens):
    mesh = plsc.VectorSubcoreMesh(core_axis_name="c", subcore_axis_name="s")
    fn = pl.kernel(
        _topk_body,
        out_type=(jax.ShapeDtypeStruct((B, K), jnp.float32),
                  jax.ShapeDtypeStruct((B, K), jnp.int32)),
        mesh=mesh,
        scratch_types=[
            pltpu.VMEM((KV,), jnp.float32),
            pltpu.VMEM((KV,), jnp.int32),
            pltpu.VMEM((KV,), jnp.int32),
            pltpu.VMEM((KV,), jnp.int32),
            pltpu.VMEM((KV,), jnp.int32),
            pltpu.VMEM((NBINS,), jnp.int32),
            pltpu.VMEM((NLANE,), jnp.int32),
            pltpu.VMEM((K,), jnp.float32),
            pltpu.VMEM((K,), jnp.int32),
        ],
        compiler_params=pltpu.CompilerParams(needs_layout_passes=False),
    )
    return fn(scores, kv_lens)


def kernel(query, weights, index_kv_cache, kv_lens, block_size, layer_id,
           index_scratch):
    scores = _scores(query, weights, index_kv_cache, index_scratch)
    lens_b = jnp.broadcast_to(
        kv_lens.astype(jnp.int32)[:, None], (B, NLANE))
    return _topk(scores, lens_b)
